# baseline (speedup 1.0000x reference)
import functools

import jax
import jax.numpy as jnp
from jax import lax
from jax.experimental import pallas as pl
from jax.experimental.pallas import tpu as pltpu
from jax.experimental.pallas import tpu_sc as plsc

_B = 16384
_C = 1000
_SMOOTH = 0.1
_NEG = _SMOOTH / _C
_POS = 1.0 - _SMOOTH + _NEG

_NC = 2
_NS = 16
_NW = _NC * _NS
_BATCH_W = _B // _NW
_BGROUPS = _BATCH_W // 16
_CHUNK_C = 120
_CHUNK_LIST = [(k * _CHUNK_C, _CHUNK_C) for k in range(_C // _CHUNK_C)]
_CHUNK_LIST.append((_C - _C % _CHUNK_C, _C % _CHUNK_C))


def _body(target_hbm, out_hbm, tgt_v, buf0, buf1, sem0, sem1):
    wid = lax.axis_index("s") * _NC + lax.axis_index("c")
    bbase = wid * _BATCH_W
    pltpu.sync_copy(target_hbm.at[pl.ds(bbase, _BATCH_W)], tgt_v)

    neg16 = jnp.full((16,), _NEG, jnp.float32)
    pos16 = jnp.full((16,), _POS, jnp.float32)
    lane = lax.iota(jnp.int32, 16)

    def fill(r, carry):
        for g in range(_BGROUPS):
            buf0[r, pl.ds(g * 16, 16)] = neg16
            buf1[r, pl.ds(g * 16, 16)] = neg16
        return carry

    lax.fori_loop(0, _CHUNK_C, fill, 0)

    def scan_pass(buf, new_c, old_c):
        c0, h = new_c

        def one(g, carry):
            col = lane + g * 16
            t = tgt_v[pl.ds(g * 16, 16)]
            if old_c is not None:
                oc0, oh = old_c
                mo = (t >= oc0) & (t < oc0 + oh)
                plsc.store_scatter(buf, [t - oc0, col], neg16, mask=mo)
            m = (t >= c0) & (t < c0 + h)
            plsc.store_scatter(buf, [t - c0, col], pos16, mask=m)
            return carry

        lax.fori_loop(0, _BGROUPS, one, 0)

    bufs = (buf0, buf1)
    sems = (sem0, sem1)
    copies = [None, None]
    for c, (c0, h) in enumerate(_CHUNK_LIST):
        b = c % 2
        buf = bufs[b]
        old_c = None
        if copies[b] is not None:
            copies[b].wait()
            old_c = _CHUNK_LIST[c - 2]
        scan_pass(buf, (c0, h), old_c)
        src = buf if h == _CHUNK_C else buf.at[pl.ds(0, h)]
        cp = pltpu.make_async_copy(
            src,
            out_hbm.at[pl.ds(c0, h), pl.ds(bbase, _BATCH_W)],
            sems[b],
        )
        cp.start()
        copies[b] = cp
    copies[0].wait()
    copies[1].wait()


_sc_call = functools.partial(
    pl.kernel,
    out_type=jax.ShapeDtypeStruct((_C, _B), jnp.float32),
    mesh=plsc.VectorSubcoreMesh(core_axis_name="c", subcore_axis_name="s"),
    compiler_params=pltpu.CompilerParams(needs_layout_passes=False),
    scratch_types=[
        pltpu.VMEM((_BATCH_W,), jnp.int32),
        pltpu.VMEM((_CHUNK_C, _BATCH_W), jnp.float32),
        pltpu.VMEM((_CHUNK_C, _BATCH_W), jnp.float32),
        pltpu.SemaphoreType.DMA,
        pltpu.SemaphoreType.DMA,
    ],
)(_body)


def kernel(target):
    return _sc_call(target).T


# speedup vs baseline: 1.0547x; 1.0547x over previous
import functools

import jax
import jax.numpy as jnp
from jax import lax
from jax.experimental import pallas as pl
from jax.experimental.pallas import tpu as pltpu
from jax.experimental.pallas import tpu_sc as plsc

_B = 16384
_C = 1000
_SMOOTH = 0.1
_NEG = _SMOOTH / _C
_POS = 1.0 - _SMOOTH + _NEG

_NC = 2
_NS = 16
_NW = _NC * _NS
_BATCH_W = _B // _NW
_BGROUPS = _BATCH_W // 16
_CHUNK_C = 40
_CHUNK_LIST = [(k * _CHUNK_C, _CHUNK_C) for k in range(_C // _CHUNK_C)]


def _body(target_hbm, out_hbm, tgt_v, buf0, buf1, sem0, sem1):
    wid = lax.axis_index("s") * _NC + lax.axis_index("c")
    bbase = wid * _BATCH_W
    pltpu.sync_copy(target_hbm.at[pl.ds(bbase, _BATCH_W)], tgt_v)

    neg16 = jnp.full((16,), _NEG, jnp.float32)
    pos16 = jnp.full((16,), _POS, jnp.float32)
    lane = lax.iota(jnp.int32, 16)

    def fill(r, carry):
        for g in range(_BGROUPS):
            buf0[r, pl.ds(g * 16, 16)] = neg16
            buf1[r, pl.ds(g * 16, 16)] = neg16
        return carry

    lax.fori_loop(0, _CHUNK_C, fill, 0)

    def scan_pass(buf, new_c, old_c):
        c0, h = new_c

        def one(g, carry):
            col = lane + g * 16
            t = tgt_v[pl.ds(g * 16, 16)]
            if old_c is not None:
                oc0, oh = old_c
                mo = (t >= oc0) & (t < oc0 + oh)
                plsc.store_scatter(buf, [t - oc0, col], neg16, mask=mo)
            m = (t >= c0) & (t < c0 + h)
            plsc.store_scatter(buf, [t - c0, col], pos16, mask=m)
            return carry

        lax.fori_loop(0, _BGROUPS, one, 0)

    bufs = (buf0, buf1)
    sems = (sem0, sem1)
    copies = [None, None]
    for c, (c0, h) in enumerate(_CHUNK_LIST):
        b = c % 2
        buf = bufs[b]
        old_c = None
        if copies[b] is not None:
            copies[b].wait()
            old_c = _CHUNK_LIST[c - 2]
        scan_pass(buf, (c0, h), old_c)
        src = buf if h == _CHUNK_C else buf.at[pl.ds(0, h)]
        cp = pltpu.make_async_copy(
            src,
            out_hbm.at[pl.ds(c0, h), pl.ds(bbase, _BATCH_W)],
            sems[b],
        )
        cp.start()
        copies[b] = cp
    copies[0].wait()
    copies[1].wait()


_sc_call = functools.partial(
    pl.kernel,
    out_type=jax.ShapeDtypeStruct((_C, _B), jnp.float32),
    mesh=plsc.VectorSubcoreMesh(core_axis_name="c", subcore_axis_name="s"),
    compiler_params=pltpu.CompilerParams(
        needs_layout_passes=False,
        disable_bounds_checks=True,
        disable_semaphore_checks=True,
        skip_device_barrier=True,
    ),
    scratch_types=[
        pltpu.VMEM((_BATCH_W,), jnp.int32),
        pltpu.VMEM((_CHUNK_C, _BATCH_W), jnp.float32),
        pltpu.VMEM((_CHUNK_C, _BATCH_W), jnp.float32),
        pltpu.SemaphoreType.DMA,
        pltpu.SemaphoreType.DMA,
    ],
)(_body)


def kernel(target):
    return _sc_call(target).T
